# baseline (speedup 1.0000x reference)
import functools

import jax
import jax.numpy as jnp
from jax import lax
from jax.experimental import pallas as pl
from jax.experimental.pallas import tpu as pltpu
from jax.experimental.pallas import tpu_sc as plsc

D_MODEL = 64
GROUP = 128


@functools.partial(jax.jit, static_argnames=("b_total",))
def _sc_embedding_lookup(x_flat_2d, table, b_total):
    info = plsc.get_sparse_core_info()
    nc, ns = info.num_cores, info.num_subcores
    nw = nc * ns
    per_w = b_total // nw
    ng = per_w // GROUP

    mesh = plsc.VectorSubcoreMesh(core_axis_name="c", subcore_axis_name="s")

    @functools.partial(
        pl.kernel,
        mesh=mesh,
        out_type=jax.ShapeDtypeStruct((b_total, D_MODEL), jnp.float32),
        scratch_types=[
            pltpu.VMEM((ng, GROUP), jnp.int32),
            pltpu.VMEM((GROUP, D_MODEL), jnp.float32),
            pltpu.SemaphoreType.DMA,
        ],
        compiler_params=pltpu.CompilerParams(use_tc_tiling_on_sc=False),
    )
    def k(x_hbm, table_hbm, out_hbm, idx_v, rows_v, sem):
        wid = lax.axis_index("s") * nc + lax.axis_index("c")
        pltpu.sync_copy(x_hbm.at[wid], idx_v)

        def body(j, carry):
            pltpu.async_copy(table_hbm.at[idx_v.at[j]], rows_v, sem).wait()
            pltpu.sync_copy(rows_v, out_hbm.at[pl.ds(wid * per_w + j * GROUP, GROUP)])
            return carry

        lax.fori_loop(0, ng, body, 0)

    return k(x_flat_2d, table)


def kernel(x, table):
    b_total = x.shape[0] * x.shape[1]
    info = plsc.get_sparse_core_info()
    nw = info.num_cores * info.num_subcores
    x_3d = x.reshape(nw, b_total // (nw * GROUP), GROUP).astype(jnp.int32)
    out = _sc_embedding_lookup(x_3d, table, b_total)
    return out.reshape(x.shape[0], x.shape[1], D_MODEL)


# speedup vs baseline: 1.0434x; 1.0434x over previous
import functools

import jax
import jax.numpy as jnp
from jax import lax
from jax.experimental import pallas as pl
from jax.experimental.pallas import tpu as pltpu
from jax.experimental.pallas import tpu_sc as plsc

D_MODEL = 64
GROUP = 128
NBUF = 5


@functools.partial(jax.jit, static_argnames=("b_total",))
def _sc_embedding_lookup(x_3d, table, b_total):
    info = plsc.get_sparse_core_info()
    nc, ns = info.num_cores, info.num_subcores
    nw = nc * ns
    per_w = b_total // nw
    ng = per_w // GROUP
    assert ng % NBUF == 0

    mesh = plsc.VectorSubcoreMesh(core_axis_name="c", subcore_axis_name="s")

    @functools.partial(
        pl.kernel,
        mesh=mesh,
        out_type=jax.ShapeDtypeStruct((b_total, D_MODEL), jnp.float32),
        scratch_types=[
            pltpu.VMEM((ng, GROUP), jnp.int32),
            pltpu.VMEM((NBUF, GROUP, D_MODEL), jnp.float32),
            [pltpu.SemaphoreType.DMA] * NBUF,
        ],
        compiler_params=pltpu.CompilerParams(use_tc_tiling_on_sc=False),
    )
    def k(x_hbm, table_hbm, out_hbm, idx_v, rows_v, sems):
        wid = lax.axis_index("s") * nc + lax.axis_index("c")
        pltpu.sync_copy(x_hbm.at[wid], idx_v)
        out_base = wid * per_w

        def gather(j, b):
            pltpu.async_copy(table_hbm.at[idx_v.at[j]], rows_v.at[b], sems[b])

        def wait_gather(j, b):
            pltpu.make_async_copy(
                table_hbm.at[idx_v.at[j]], rows_v.at[b], sems[b]
            ).wait()

        def write_out(j, b):
            pltpu.sync_copy(rows_v.at[b], out_hbm.at[pl.ds(out_base + j * GROUP, GROUP)])

        for b in range(NBUF):
            gather(b, b)

        def outer_body(t, carry):
            j0 = t * NBUF
            for b in range(NBUF):
                j = j0 + b
                wait_gather(j, b)
                write_out(j, b)
                gather(j + NBUF, b)
            return carry

        lax.fori_loop(0, ng // NBUF - 1, outer_body, 0)

        j0 = ng - NBUF
        for b in range(NBUF):
            j = j0 + b
            wait_gather(j, b)
            write_out(j, b)

    return k(x_3d, table)


def kernel(x, table):
    b_total = x.shape[0] * x.shape[1]
    info = plsc.get_sparse_core_info()
    nw = info.num_cores * info.num_subcores
    x_3d = x.reshape(nw, b_total // (nw * GROUP), GROUP).astype(jnp.int32)
    out = _sc_embedding_lookup(x_3d, table, b_total)
    return out.reshape(x.shape[0], x.shape[1], D_MODEL)
